# Initial kernel scaffold; baseline (speedup 1.0000x reference)
import jax
import jax.numpy as jnp
from jax.experimental import pallas as pl


def kernel(adj_indices, adj_values, features, W1, b1, W2, b2, W3, b3, W_fc, b_fc):
    raise NotImplementedError("write your pallas kernel here")


# speedup vs baseline: 9.5736x; 9.5736x over previous
import functools

import jax
import jax.numpy as jnp
from jax import lax
from jax.experimental import pallas as pl
from jax.experimental.pallas import tpu as pltpu
from jax.experimental.pallas import tpu_sc as plsc

N = 10000
E = 320000
D = 128
H3 = 192
NC = 2
NS = 16
NW = NC * NS
EPW = E // NW
CH = 80
NCH = EPW // CH
RPS = N // NS

_f32 = jnp.float32
_i32 = jnp.int32


def _dense_body(x_ref, wcat_ref, bcat_ref, wa_ref, wb_ref, p1_ref, p23_ref):
    h = jnp.dot(x_ref[...], wcat_ref[...], preferred_element_type=_f32)
    h = jnp.maximum(h + bcat_ref[...], 0.0)
    p1_ref[...] = jnp.dot(h, wa_ref[...], preferred_element_type=_f32)
    p23_ref[...] = jnp.dot(h, wb_ref[...], preferred_element_type=_f32)


def _dense_stage(x, wcat, bcat, wa, wb):
    blk = 1000
    grid = N // blk
    return pl.pallas_call(
        _dense_body,
        grid=(grid,),
        in_specs=[
            pl.BlockSpec((blk, D), lambda i: (i, 0)),
            pl.BlockSpec((D, H3), lambda i: (0, 0)),
            pl.BlockSpec((1, H3), lambda i: (0, 0)),
            pl.BlockSpec((H3, 16), lambda i: (0, 0)),
            pl.BlockSpec((H3, 32), lambda i: (0, 0)),
        ],
        out_specs=[
            pl.BlockSpec((blk, 16), lambda i: (i, 0)),
            pl.BlockSpec((blk, 32), lambda i: (i, 0)),
        ],
        out_shape=[
            jax.ShapeDtypeStruct((N, 16), _f32),
            jax.ShapeDtypeStruct((N, 32), _f32),
        ],
    )(x, wcat, bcat, wa, wb)


def _spmm2_body(rows_hbm, cols_hbm, vals_hbm, p23_hbm, zero_hbm,
                qa_hbm, tp_hbm,
                acc_q, acc_t, rowv, colv, valv, g, gq, gt, sem):
    c = lax.axis_index("c")
    s = lax.axis_index("s")
    wid = c * NS + s
    pltpu.sync_copy(zero_hbm.at[pl.ds(s * RPS, RPS)], acc_q.at[pl.ds(s * RPS, RPS)])
    pltpu.sync_copy(zero_hbm.at[pl.ds(s * RPS, RPS)], acc_t.at[pl.ds(s * RPS, RPS)])
    pltpu.sync_copy(rows_hbm.at[wid], rowv)
    pltpu.sync_copy(cols_hbm.at[wid], colv)
    pltpu.sync_copy(vals_hbm.at[wid], valv)
    plsc.subcore_barrier()

    def chunk(j, carry):
        pltpu.async_copy(p23_hbm.at[colv.at[j]], g, sem).wait()
        for i in range(CH):
            vi = plsc.load_gather(
                valv,
                [jnp.full((16,), j, _i32), jnp.full((16,), i, _i32)],
            )
            gq[i, :] = g[i, pl.ds(0, 16)] * vi
            gt[i, :] = g[i, pl.ds(16, 16)] * vi
        pltpu.sync_copy(gq, acc_q.at[rowv.at[j]], add=True)
        pltpu.sync_copy(gt, acc_t.at[rowv.at[j]], add=True)
        return carry

    lax.fori_loop(0, NCH, chunk, 0)
    plsc.subcore_barrier()
    pltpu.sync_copy(acc_q.at[pl.ds(s * RPS, RPS)], qa_hbm.at[c, pl.ds(s * RPS, RPS)])
    pltpu.sync_copy(acc_t.at[pl.ds(s * RPS, RPS)], tp_hbm.at[c, pl.ds(s * RPS, RPS)])


def _spmm2_stage(rows3, cols3, vals3, p23, zero):
    mesh = plsc.VectorSubcoreMesh(core_axis_name="c", subcore_axis_name="s")
    f = pl.kernel(
        _spmm2_body,
        out_type=[
            jax.ShapeDtypeStruct((NC, N, 16), _f32),
            jax.ShapeDtypeStruct((NC, N, 16), _f32),
        ],
        mesh=mesh,
        compiler_params=pltpu.CompilerParams(use_tc_tiling_on_sc=False, needs_layout_passes=False),
        scratch_types=[
            pltpu.VMEM_SHARED((N, 16), _f32),
            pltpu.VMEM_SHARED((N, 16), _f32),
            pltpu.VMEM((NCH, CH), _i32),
            pltpu.VMEM((NCH, CH), _i32),
            pltpu.VMEM((NCH, CH), _f32),
            pltpu.VMEM((CH, 32), _f32),
            pltpu.VMEM((CH, 16), _f32),
            pltpu.VMEM((CH, 16), _f32),
            pltpu.SemaphoreType.DMA,
        ],
    )
    return f(rows3, cols3, vals3, p23, zero)


def _spmm1_body(rows_hbm, cols_hbm, vals_hbm, t_hbm, zero_hbm,
                qb_hbm,
                acc, rowv, colv, valv, g, gq, sem):
    c = lax.axis_index("c")
    s = lax.axis_index("s")
    wid = c * NS + s
    pltpu.sync_copy(zero_hbm.at[pl.ds(s * RPS, RPS)], acc.at[pl.ds(s * RPS, RPS)])
    pltpu.sync_copy(rows_hbm.at[wid], rowv)
    pltpu.sync_copy(cols_hbm.at[wid], colv)
    pltpu.sync_copy(vals_hbm.at[wid], valv)
    plsc.subcore_barrier()

    def chunk(j, carry):
        pltpu.async_copy(t_hbm.at[colv.at[j]], g, sem).wait()
        for i in range(CH):
            vi = plsc.load_gather(
                valv,
                [jnp.full((16,), j, _i32), jnp.full((16,), i, _i32)],
            )
            gq[i, :] = g[i, :] * vi
        pltpu.sync_copy(gq, acc.at[rowv.at[j]], add=True)
        return carry

    lax.fori_loop(0, NCH, chunk, 0)
    plsc.subcore_barrier()
    pltpu.sync_copy(acc.at[pl.ds(s * RPS, RPS)], qb_hbm.at[c, pl.ds(s * RPS, RPS)])


def _spmm1_stage(rows3, cols3, vals3, t, zero):
    mesh = plsc.VectorSubcoreMesh(core_axis_name="c", subcore_axis_name="s")
    f = pl.kernel(
        _spmm1_body,
        out_type=jax.ShapeDtypeStruct((NC, N, 16), _f32),
        mesh=mesh,
        compiler_params=pltpu.CompilerParams(use_tc_tiling_on_sc=False, needs_layout_passes=False),
        scratch_types=[
            pltpu.VMEM_SHARED((N, 16), _f32),
            pltpu.VMEM((NCH, CH), _i32),
            pltpu.VMEM((NCH, CH), _i32),
            pltpu.VMEM((NCH, CH), _f32),
            pltpu.VMEM((CH, 16), _f32),
            pltpu.VMEM((CH, 16), _f32),
            pltpu.SemaphoreType.DMA,
        ],
    )
    return f(rows3, cols3, vals3, t, zero)


def _combine_body(tp_ref, t_ref):
    t_ref[...] = tp_ref[0] + tp_ref[1]


def _combine_stage(tp):
    blk = 1000
    return pl.pallas_call(
        _combine_body,
        grid=(N // blk,),
        in_specs=[pl.BlockSpec((NC, blk, 16), lambda i: (0, i, 0))],
        out_specs=pl.BlockSpec((blk, 16), lambda i: (i, 0)),
        out_shape=jax.ShapeDtypeStruct((N, 16), _f32),
    )(tp)


def _final_body(p1_ref, qa_ref, qb_ref, bfc_ref, out_ref):
    logits = (p1_ref[...] + qa_ref[0] + qa_ref[1] + qb_ref[0] + qb_ref[1]
              + bfc_ref[...])
    m = jnp.max(logits, axis=1, keepdims=True)
    sh = logits - m
    lse = jnp.log(jnp.sum(jnp.exp(sh), axis=1, keepdims=True))
    out_ref[...] = sh - lse


def _final_stage(p1, qa, qb, bfc):
    blk = 1000
    return pl.pallas_call(
        _final_body,
        grid=(N // blk,),
        in_specs=[
            pl.BlockSpec((blk, 16), lambda i: (i, 0)),
            pl.BlockSpec((NC, blk, 16), lambda i: (0, i, 0)),
            pl.BlockSpec((NC, blk, 16), lambda i: (0, i, 0)),
            pl.BlockSpec((1, 16), lambda i: (0, 0)),
        ],
        out_specs=pl.BlockSpec((blk, 16), lambda i: (i, 0)),
        out_shape=jax.ShapeDtypeStruct((N, 16), _f32),
    )(p1, qa, qb, bfc)


def kernel(adj_indices, adj_values, features, W1, b1, W2, b2, W3, b3,
           W_fc, b_fc):
    rows3 = adj_indices[0].reshape(NW, NCH, CH)
    cols3 = adj_indices[1].reshape(NW, NCH, CH)
    vals3 = adj_values.reshape(NW, NCH, CH)

    wcat = jnp.concatenate([W1, W2, W3], axis=1)
    bcat = jnp.concatenate([b1, b2, b3]).reshape(1, H3)
    z64 = jnp.zeros((64, 16), _f32)
    wa = jnp.concatenate([W_fc[0:64], z64, z64], axis=0)
    wb = jnp.zeros((H3, 32), _f32)
    wb = wb.at[64:128, 0:16].set(W_fc[64:128])
    wb = wb.at[128:192, 16:32].set(W_fc[128:192])
    zero = jnp.zeros((N, 16), _f32)
    bfc = b_fc.reshape(1, 16)

    p1, p23 = _dense_stage(features, wcat, bcat, wa, wb)
    qa, tp = _spmm2_stage(rows3, cols3, vals3, p23, zero)
    t = _combine_stage(tp)
    qb = _spmm1_stage(rows3, cols3, vals3, t, zero)
    return _final_stage(p1, qa, qb, bfc)


# speedup vs baseline: 10.6055x; 1.1078x over previous
import jax
import jax.numpy as jnp
from jax import lax
from jax.experimental import pallas as pl
from jax.experimental.pallas import tpu as pltpu
from jax.experimental.pallas import tpu_sc as plsc

N = 10000
E = 320000
D = 128
H3 = 192
NC = 2
NS = 16
NW = NC * NS
CH = 128
NCH = 80
EPP = NCH * CH
RPS = N // NS

_f32 = jnp.float32
_i32 = jnp.int32

_SC_PARAMS = pltpu.CompilerParams(
    use_tc_tiling_on_sc=False, needs_layout_passes=False)


def _dense_body(x_ref, wcat_ref, bcat_ref, wa_ref, wb_ref, p1_ref, p23_ref):
    h = jnp.dot(x_ref[...], wcat_ref[...], preferred_element_type=_f32)
    h = jnp.maximum(h + bcat_ref[...], 0.0)
    p1_ref[...] = jnp.dot(h, wa_ref[...], preferred_element_type=_f32)
    p23_ref[...] = jnp.dot(h, wb_ref[...], preferred_element_type=_f32)


def _dense_stage(x, wcat, bcat, wa, wb):
    blk = 1000
    return pl.pallas_call(
        _dense_body,
        grid=(N // blk,),
        in_specs=[
            pl.BlockSpec((blk, D), lambda i: (i, 0)),
            pl.BlockSpec((D, H3), lambda i: (0, 0)),
            pl.BlockSpec((1, H3), lambda i: (0, 0)),
            pl.BlockSpec((H3, 16), lambda i: (0, 0)),
            pl.BlockSpec((H3, 32), lambda i: (0, 0)),
        ],
        out_specs=[
            pl.BlockSpec((blk, 16), lambda i: (i, 0)),
            pl.BlockSpec((blk, 32), lambda i: (i, 0)),
        ],
        out_shape=[
            jax.ShapeDtypeStruct((N, 16), _f32),
            jax.ShapeDtypeStruct((N, 32), _f32),
        ],
    )(x, wcat, bcat, wa, wb)


def _make_spmm_body(w):
    def body(rows_hbm, cols_hbm, vals_hbm, tab_hbm, zero_hbm, out_hbm,
             acc, rowv, colv, valv, g0, g1, sq0, sq1, gsem0, gsem1):
        c = lax.axis_index("c")
        s = lax.axis_index("s")
        wid = c * NS + s
        pltpu.sync_copy(zero_hbm.at[pl.ds(s * RPS, RPS)],
                        acc.at[pl.ds(s * RPS, RPS)])
        pltpu.sync_copy(rows_hbm.at[wid], rowv)
        pltpu.sync_copy(cols_hbm.at[wid], colv)
        pltpu.sync_copy(vals_hbm.at[wid], valv)
        plsc.subcore_barrier()

        def scale(j, g, sq):
            for i in range(CH):
                vi = plsc.load_gather(
                    valv,
                    [jnp.full((16,), j, _i32), jnp.full((16,), i, _i32)],
                )
                for h in range(w // 16):
                    sq[i, pl.ds(16 * h, 16)] = g[i, pl.ds(16 * h, 16)] * vi

        def process(j, g, sq, gsem):
            pltpu.make_async_copy(tab_hbm.at[colv.at[j]], g, gsem).wait()
            scale(j, g, sq)
            pltpu.sync_copy(sq, acc.at[rowv.at[j]], add=True)

        def start_gather(j, g, gsem):
            pltpu.async_copy(tab_hbm.at[colv.at[j]], g, gsem)

        start_gather(0, g0, gsem0)
        start_gather(1, g1, gsem1)

        def pair(k, carry):
            j0 = 2 * k
            process(j0, g0, sq0, gsem0)
            start_gather(j0 + 2, g0, gsem0)
            process(j0 + 1, g1, sq1, gsem1)
            start_gather(j0 + 3, g1, gsem1)
            return carry

        lax.fori_loop(0, NCH // 2 - 1, pair, 0)
        process(NCH - 2, g0, sq0, gsem0)
        process(NCH - 1, g1, sq1, gsem1)

        plsc.subcore_barrier()
        pltpu.sync_copy(acc.at[pl.ds(s * RPS, RPS)],
                        out_hbm.at[c, pl.ds(s * RPS, RPS)])

    return body


def _spmm_stage(rows3, cols3, vals3, tab, zero, w):
    mesh = plsc.VectorSubcoreMesh(core_axis_name="c", subcore_axis_name="s")
    f = pl.kernel(
        _make_spmm_body(w),
        out_type=jax.ShapeDtypeStruct((NC, N, w), _f32),
        mesh=mesh,
        compiler_params=_SC_PARAMS,
        scratch_types=[
            pltpu.VMEM_SHARED((N, w), _f32),
            pltpu.VMEM((NCH, CH), _i32),
            pltpu.VMEM((NCH, CH), _i32),
            pltpu.VMEM((NCH, CH), _f32),
            pltpu.VMEM((CH, w), _f32),
            pltpu.VMEM((CH, w), _f32),
            pltpu.VMEM((CH, w), _f32),
            pltpu.VMEM((CH, w), _f32),
            pltpu.SemaphoreType.DMA,
            pltpu.SemaphoreType.DMA,
        ],
    )
    return f(rows3, cols3, vals3, tab, zero)


def _combine_body(tp_ref, t_ref):
    t_ref[...] = tp_ref[0, :, 16:32] + tp_ref[1, :, 16:32]


def _combine_stage(outa):
    blk = 1000
    return pl.pallas_call(
        _combine_body,
        grid=(N // blk,),
        in_specs=[pl.BlockSpec((NC, blk, 32), lambda i: (0, i, 0))],
        out_specs=pl.BlockSpec((blk, 16), lambda i: (i, 0)),
        out_shape=jax.ShapeDtypeStruct((N, 16), _f32),
    )(outa)


def _final_body(p1_ref, qa_ref, qb_ref, bfc_ref, out_ref):
    logits = (p1_ref[...] + qa_ref[0, :, 0:16] + qa_ref[1, :, 0:16]
              + qb_ref[0] + qb_ref[1] + bfc_ref[...])
    m = jnp.max(logits, axis=1, keepdims=True)
    sh = logits - m
    lse = jnp.log(jnp.sum(jnp.exp(sh), axis=1, keepdims=True))
    out_ref[...] = sh - lse


def _final_stage(p1, outa, outb, bfc):
    blk = 1000
    return pl.pallas_call(
        _final_body,
        grid=(N // blk,),
        in_specs=[
            pl.BlockSpec((blk, 16), lambda i: (i, 0)),
            pl.BlockSpec((NC, blk, 32), lambda i: (0, i, 0)),
            pl.BlockSpec((NC, blk, 16), lambda i: (0, i, 0)),
            pl.BlockSpec((1, 16), lambda i: (0, 0)),
        ],
        out_specs=pl.BlockSpec((blk, 16), lambda i: (i, 0)),
        out_shape=jax.ShapeDtypeStruct((N, 16), _f32),
    )(p1, outa, outb, bfc)


def kernel(adj_indices, adj_values, features, W1, b1, W2, b2, W3, b3,
           W_fc, b_fc):
    epw = E // NW
    pad = EPP - epw
    rows3 = jnp.pad(adj_indices[0].reshape(NW, epw), ((0, 0), (0, pad))
                    ).reshape(NW, NCH, CH)
    cols3 = jnp.pad(adj_indices[1].reshape(NW, epw), ((0, 0), (0, pad))
                    ).reshape(NW, NCH, CH)
    vals3 = jnp.pad(adj_values.reshape(NW, epw), ((0, 0), (0, pad))
                    ).reshape(NW, NCH, CH)

    wcat = jnp.concatenate([W1, W2, W3], axis=1)
    bcat = jnp.concatenate([b1, b2, b3]).reshape(1, H3)
    z64 = jnp.zeros((64, 16), _f32)
    wa = jnp.concatenate([W_fc[0:64], z64, z64], axis=0)
    wb = jnp.zeros((H3, 32), _f32)
    wb = wb.at[64:128, 0:16].set(W_fc[64:128])
    wb = wb.at[128:192, 16:32].set(W_fc[128:192])
    zero32 = jnp.zeros((N, 32), _f32)
    zero16 = jnp.zeros((N, 16), _f32)
    bfc = b_fc.reshape(1, 16)

    p1, p23 = _dense_stage(features, wcat, bcat, wa, wb)
    outa = _spmm_stage(rows3, cols3, vals3, p23, zero32, 32)
    t = _combine_stage(outa)
    outb = _spmm_stage(rows3, cols3, vals3, t, zero16, 16)
    return _final_stage(p1, outa, outb, bfc)


# speedup vs baseline: 12.9266x; 1.2189x over previous
import jax
import jax.numpy as jnp
from jax import lax
from jax.experimental import pallas as pl
from jax.experimental.pallas import tpu as pltpu
from jax.experimental.pallas import tpu_sc as plsc

N = 10000
E = 320000
D = 128
H3 = 192
NC = 2
NS = 16
NW = NC * NS
CH = 128
NCH = 80
EPP = NCH * CH
RPS = N // NS

_f32 = jnp.float32
_i32 = jnp.int32

_SC_PARAMS = pltpu.CompilerParams(
    use_tc_tiling_on_sc=False, needs_layout_passes=False)


def _dense_body(x_ref, wcat_ref, bcat_ref, wa_ref, wb_ref, wc_ref,
                p1_ref, p2_ref, p3_ref):
    h = jnp.dot(x_ref[...], wcat_ref[...], preferred_element_type=_f32)
    h = jnp.maximum(h + bcat_ref[...], 0.0)
    p1_ref[...] = jnp.dot(h, wa_ref[...], preferred_element_type=_f32)
    p2_ref[...] = jnp.dot(h, wb_ref[...], preferred_element_type=_f32)
    p3_ref[...] = jnp.dot(h, wc_ref[...], preferred_element_type=_f32)


def _dense_stage(x, wcat, bcat, wa, wb, wc):
    blk = 1000
    return pl.pallas_call(
        _dense_body,
        grid=(N // blk,),
        in_specs=[
            pl.BlockSpec((blk, D), lambda i: (i, 0)),
            pl.BlockSpec((D, H3), lambda i: (0, 0)),
            pl.BlockSpec((1, H3), lambda i: (0, 0)),
            pl.BlockSpec((H3, 16), lambda i: (0, 0)),
            pl.BlockSpec((H3, 16), lambda i: (0, 0)),
            pl.BlockSpec((H3, 16), lambda i: (0, 0)),
        ],
        out_specs=[
            pl.BlockSpec((blk, 16), lambda i: (i, 0)),
            pl.BlockSpec((blk, 16), lambda i: (i, 0)),
            pl.BlockSpec((blk, 16), lambda i: (i, 0)),
        ],
        out_shape=[
            jax.ShapeDtypeStruct((N, 16), _f32),
            jax.ShapeDtypeStruct((N, 16), _f32),
            jax.ShapeDtypeStruct((N, 16), _f32),
        ],
    )(x, wcat, bcat, wa, wb, wc)


def _make_spmm_body(w):
    def body(rows_hbm, cols_hbm, vals_hbm, tab_hbm, zero_hbm, out_hbm,
             acc, rowv, colv, valv, g0, g1, sq0, sq1, gsem0, gsem1):
        c = lax.axis_index("c")
        s = lax.axis_index("s")
        wid = c * NS + s
        pltpu.sync_copy(zero_hbm.at[pl.ds(s * RPS, RPS)],
                        acc.at[pl.ds(s * RPS, RPS)])
        pltpu.sync_copy(rows_hbm.at[wid], rowv)
        pltpu.sync_copy(cols_hbm.at[wid], colv)
        pltpu.sync_copy(vals_hbm.at[wid], valv)
        plsc.subcore_barrier()

        def scale(j, g, sq):
            for i in range(CH):
                vi = plsc.load_gather(
                    valv,
                    [jnp.full((16,), j, _i32), jnp.full((16,), i, _i32)],
                )
                for h in range(w // 16):
                    sq[i, pl.ds(16 * h, 16)] = g[i, pl.ds(16 * h, 16)] * vi

        def process(j, g, sq, gsem):
            pltpu.make_async_copy(tab_hbm.at[colv.at[j]], g, gsem).wait()
            scale(j, g, sq)
            pltpu.sync_copy(sq, acc.at[rowv.at[j]], add=True)

        def start_gather(j, g, gsem):
            pltpu.async_copy(tab_hbm.at[colv.at[j]], g, gsem)

        start_gather(0, g0, gsem0)
        start_gather(1, g1, gsem1)

        def pair(k, carry):
            j0 = 2 * k
            process(j0, g0, sq0, gsem0)
            start_gather(j0 + 2, g0, gsem0)
            process(j0 + 1, g1, sq1, gsem1)
            start_gather(j0 + 3, g1, gsem1)
            return carry

        lax.fori_loop(0, NCH // 2 - 1, pair, 0)
        process(NCH - 2, g0, sq0, gsem0)
        process(NCH - 1, g1, sq1, gsem1)

        plsc.subcore_barrier()
        pltpu.sync_copy(acc.at[pl.ds(s * RPS, RPS)],
                        out_hbm.at[c, pl.ds(s * RPS, RPS)])

    return body


def _spmm_stage(rows3, cols3, vals3, tab, zero, w):
    mesh = plsc.VectorSubcoreMesh(core_axis_name="c", subcore_axis_name="s")
    f = pl.kernel(
        _make_spmm_body(w),
        out_type=jax.ShapeDtypeStruct((NC, N, w), _f32),
        mesh=mesh,
        compiler_params=_SC_PARAMS,
        scratch_types=[
            pltpu.VMEM_SHARED((N, w), _f32),
            pltpu.VMEM((NCH, CH), _i32),
            pltpu.VMEM((NCH, CH), _i32),
            pltpu.VMEM((NCH, CH), _f32),
            pltpu.VMEM((CH, w), _f32),
            pltpu.VMEM((CH, w), _f32),
            pltpu.VMEM((CH, w), _f32),
            pltpu.VMEM((CH, w), _f32),
            pltpu.SemaphoreType.DMA,
            pltpu.SemaphoreType.DMA,
        ],
    )
    return f(rows3, cols3, vals3, tab, zero)


def _combine_body(p2_ref, tp_ref, t_ref):
    t_ref[...] = p2_ref[...] + tp_ref[0] + tp_ref[1]


def _combine_stage(p2, outa):
    blk = 1000
    return pl.pallas_call(
        _combine_body,
        grid=(N // blk,),
        in_specs=[
            pl.BlockSpec((blk, 16), lambda i: (i, 0)),
            pl.BlockSpec((NC, blk, 16), lambda i: (0, i, 0)),
        ],
        out_specs=pl.BlockSpec((blk, 16), lambda i: (i, 0)),
        out_shape=jax.ShapeDtypeStruct((N, 16), _f32),
    )(p2, outa)


def _final_body(p1_ref, qb_ref, bfc_ref, out_ref):
    logits = p1_ref[...] + qb_ref[0] + qb_ref[1] + bfc_ref[...]
    m = jnp.max(logits, axis=1, keepdims=True)
    sh = logits - m
    lse = jnp.log(jnp.sum(jnp.exp(sh), axis=1, keepdims=True))
    out_ref[...] = sh - lse


def _final_stage(p1, outb, bfc):
    blk = 1000
    return pl.pallas_call(
        _final_body,
        grid=(N // blk,),
        in_specs=[
            pl.BlockSpec((blk, 16), lambda i: (i, 0)),
            pl.BlockSpec((NC, blk, 16), lambda i: (0, i, 0)),
            pl.BlockSpec((1, 16), lambda i: (0, 0)),
        ],
        out_specs=pl.BlockSpec((blk, 16), lambda i: (i, 0)),
        out_shape=jax.ShapeDtypeStruct((N, 16), _f32),
    )(p1, outb, bfc)


def kernel(adj_indices, adj_values, features, W1, b1, W2, b2, W3, b3,
           W_fc, b_fc):
    epw = E // NW
    pad = EPP - epw
    rows3 = jnp.pad(adj_indices[0].reshape(NW, epw), ((0, 0), (0, pad))
                    ).reshape(NW, NCH, CH)
    cols3 = jnp.pad(adj_indices[1].reshape(NW, epw), ((0, 0), (0, pad))
                    ).reshape(NW, NCH, CH)
    vals3 = jnp.pad(adj_values.reshape(NW, epw), ((0, 0), (0, pad))
                    ).reshape(NW, NCH, CH)

    wcat = jnp.concatenate([W1, W2, W3], axis=1)
    bcat = jnp.concatenate([b1, b2, b3]).reshape(1, H3)
    z64 = jnp.zeros((64, 16), _f32)
    wa = jnp.concatenate([W_fc[0:64], z64, z64], axis=0)
    wb = jnp.concatenate([z64, W_fc[64:128], z64], axis=0)
    wc = jnp.concatenate([z64, z64, W_fc[128:192]], axis=0)
    zero16 = jnp.zeros((N, 16), _f32)
    bfc = b_fc.reshape(1, 16)

    p1, p2, p3 = _dense_stage(features, wcat, bcat, wa, wb, wc)
    outa = _spmm_stage(rows3, cols3, vals3, p3, zero16, 16)
    u = _combine_stage(p2, outa)
    outb = _spmm_stage(rows3, cols3, vals3, u, zero16, 16)
    return _final_stage(p1, outb, bfc)


# speedup vs baseline: 17.0787x; 1.3212x over previous
import jax
import jax.numpy as jnp
from jax import lax
from jax.experimental import pallas as pl
from jax.experimental.pallas import tpu as pltpu
from jax.experimental.pallas import tpu_sc as plsc

N = 10000
E = 320000
D = 128
NC = 2
NS = 16
NW = NC * NS
EPW = E // NW
CH = 80
NCH = EPW // CH
RPS = N // NS

_f32 = jnp.float32
_i32 = jnp.int32

_SC_PARAMS = pltpu.CompilerParams(
    use_tc_tiling_on_sc=False, needs_layout_passes=False)


def _dense_body(x_ref, w1_ref, b1_ref, w2_ref, b2_ref, w3_ref, b3_ref,
                wfc_ref, p1_ref, p2_ref, p3_ref):
    x = x_ref[...]
    for k, (w_ref, b_ref, p_ref) in enumerate(
            [(w1_ref, b1_ref, p1_ref), (w2_ref, b2_ref, p2_ref),
             (w3_ref, b3_ref, p3_ref)]):
        h = jnp.dot(x, w_ref[...], preferred_element_type=_f32)
        h = jnp.maximum(h + b_ref[...], 0.0)
        f = wfc_ref[pl.ds(64 * k, 64), :]
        p_ref[...] = jnp.dot(h, f, preferred_element_type=_f32)


def _dense_stage(x, w1, b1, w2, b2, w3, b3, wfc):
    blk = 2000
    wspec = pl.BlockSpec((D, 64), lambda i: (0, 0))
    bspec = pl.BlockSpec((1, 64), lambda i: (0, 0))
    ospec = pl.BlockSpec((blk, 16), lambda i: (i, 0))
    oshape = jax.ShapeDtypeStruct((N, 16), _f32)
    return pl.pallas_call(
        _dense_body,
        grid=(N // blk,),
        in_specs=[
            pl.BlockSpec((blk, D), lambda i: (i, 0)),
            wspec, bspec, wspec, bspec, wspec, bspec,
            pl.BlockSpec((192, 16), lambda i: (0, 0)),
        ],
        out_specs=[ospec, ospec, ospec],
        out_shape=[oshape, oshape, oshape],
    )(x, w1, b1, w2, b2, w3, b3, wfc)


def _stage_edges(adj_hbm, vals_hbm, wid, rowv, colv, valv):
    pltpu.sync_copy(adj_hbm.at[0, wid], rowv)
    pltpu.sync_copy(adj_hbm.at[1, wid], colv)
    pltpu.sync_copy(vals_hbm.at[wid], valv)


def _edge_loop(tab_hbm, acc, rowv, colv, valv, g0, g1, sq0, sq1,
               gsem0, gsem1):
    def scale(j, g, sq):
        for i in range(CH):
            vi = plsc.load_gather(
                valv,
                [jnp.full((16,), j, _i32), jnp.full((16,), i, _i32)],
            )
            sq[i, :] = g[i, :] * vi

    def process(j, g, sq, gsem):
        pltpu.make_async_copy(tab_hbm.at[colv.at[j]], g, gsem).wait()
        scale(j, g, sq)
        pltpu.sync_copy(sq, acc.at[rowv.at[j]], add=True)

    def start_gather(j, g, gsem):
        pltpu.async_copy(tab_hbm.at[colv.at[j]], g, gsem)

    start_gather(0, g0, gsem0)
    start_gather(1, g1, gsem1)

    def pair(k, carry):
        j0 = 2 * k
        process(j0, g0, sq0, gsem0)
        start_gather(j0 + 2, g0, gsem0)
        process(j0 + 1, g1, sq1, gsem1)
        start_gather(j0 + 3, g1, gsem1)
        return carry

    lax.fori_loop(0, (NCH - 3) // 2, pair, 0)
    process(NCH - 3, g0, sq0, gsem0)
    start_gather(NCH - 1, g0, gsem0)
    process(NCH - 2, g1, sq1, gsem1)
    process(NCH - 1, g0, sq0, gsem0)


def _spmm_a_body(adj_hbm, vals_hbm, tab_hbm, zero_hbm, out_hbm,
                 acc, rowv, colv, valv, g0, g1, sq0, sq1, gsem0, gsem1):
    c = lax.axis_index("c")
    s = lax.axis_index("s")
    pltpu.sync_copy(zero_hbm.at[pl.ds(s * RPS, RPS)],
                    acc.at[pl.ds(s * RPS, RPS)])
    _stage_edges(adj_hbm, vals_hbm, c * NS + s, rowv, colv, valv)
    plsc.subcore_barrier()
    _edge_loop(tab_hbm, acc, rowv, colv, valv, g0, g1, sq0, sq1,
               gsem0, gsem1)
    plsc.subcore_barrier()
    pltpu.sync_copy(acc.at[pl.ds(s * RPS, RPS)],
                    out_hbm.at[c, pl.ds(s * RPS, RPS)])


def _spmm_b_body(adj_hbm, vals_hbm, p2_hbm, pa_hbm, zero_hbm,
                 out_hbm, u_hbm,
                 acc, rowv, colv, valv, g0, g1, sq0, sq1, ub, t0b, t1b,
                 gsem0, gsem1):
    c = lax.axis_index("c")
    s = lax.axis_index("s")
    sl = pl.ds(s * RPS, RPS)
    pltpu.sync_copy(p2_hbm.at[sl], ub)
    pltpu.sync_copy(pa_hbm.at[0, sl], t0b)
    pltpu.sync_copy(pa_hbm.at[1, sl], t1b)

    def add_row(r, carry):
        ub[r, :] = ub[r, :] + t0b[r, :] + t1b[r, :]
        return carry

    lax.fori_loop(0, RPS, add_row, 0)
    pltpu.sync_copy(ub, u_hbm.at[sl])
    pltpu.sync_copy(zero_hbm.at[sl], acc.at[sl])
    _stage_edges(adj_hbm, vals_hbm, c * NS + s, rowv, colv, valv)
    plsc.subcore_barrier()
    _edge_loop(u_hbm, acc, rowv, colv, valv, g0, g1, sq0, sq1,
               gsem0, gsem1)
    plsc.subcore_barrier()
    pltpu.sync_copy(acc.at[sl], out_hbm.at[c, sl])


_EDGE_SCRATCH = [
    pltpu.VMEM_SHARED((N, 16), _f32),
    pltpu.VMEM((NCH, CH), _i32),
    pltpu.VMEM((NCH, CH), _i32),
    pltpu.VMEM((NCH, CH), _f32),
    pltpu.VMEM((CH, 16), _f32),
    pltpu.VMEM((CH, 16), _f32),
    pltpu.VMEM((CH, 16), _f32),
    pltpu.VMEM((CH, 16), _f32),
]
_SEMS = [pltpu.SemaphoreType.DMA, pltpu.SemaphoreType.DMA]


def _spmm_a_stage(adj3, vals3, tab, zero):
    mesh = plsc.VectorSubcoreMesh(core_axis_name="c", subcore_axis_name="s")
    f = pl.kernel(
        _spmm_a_body,
        out_type=jax.ShapeDtypeStruct((NC, N, 16), _f32),
        mesh=mesh,
        compiler_params=_SC_PARAMS,
        scratch_types=_EDGE_SCRATCH + _SEMS,
    )
    return f(adj3, vals3, tab, zero)


def _spmm_b_stage(adj3, vals3, p2, pa, zero):
    mesh = plsc.VectorSubcoreMesh(core_axis_name="c", subcore_axis_name="s")
    f = pl.kernel(
        _spmm_b_body,
        out_type=[
            jax.ShapeDtypeStruct((NC, N, 16), _f32),
            jax.ShapeDtypeStruct((N, 16), _f32),
        ],
        mesh=mesh,
        compiler_params=_SC_PARAMS,
        scratch_types=_EDGE_SCRATCH + [
            pltpu.VMEM((RPS, 16), _f32),
            pltpu.VMEM((RPS, 16), _f32),
            pltpu.VMEM((RPS, 16), _f32),
        ] + _SEMS,
    )
    return f(adj3, vals3, p2, pa, zero)


def _final_body(p1_ref, qb_ref, bfc_ref, out_ref):
    logits = p1_ref[...] + qb_ref[0] + qb_ref[1] + bfc_ref[...]
    m = jnp.max(logits, axis=1, keepdims=True)
    sh = logits - m
    lse = jnp.log(jnp.sum(jnp.exp(sh), axis=1, keepdims=True))
    out_ref[...] = sh - lse


def _final_stage(p1, outb, bfc):
    blk = 2000
    return pl.pallas_call(
        _final_body,
        grid=(N // blk,),
        in_specs=[
            pl.BlockSpec((blk, 16), lambda i: (i, 0)),
            pl.BlockSpec((NC, blk, 16), lambda i: (0, i, 0)),
            pl.BlockSpec((1, 16), lambda i: (0, 0)),
        ],
        out_specs=pl.BlockSpec((blk, 16), lambda i: (i, 0)),
        out_shape=jax.ShapeDtypeStruct((N, 16), _f32),
    )(p1, outb, bfc)


def kernel(adj_indices, adj_values, features, W1, b1, W2, b2, W3, b3,
           W_fc, b_fc):
    adj3 = adj_indices.reshape(2, NW, NCH, CH)
    vals3 = adj_values.reshape(NW, NCH, CH)
    zero = jnp.zeros((N, 16), _f32)
    bfc = b_fc.reshape(1, 16)

    p1, p2, p3 = _dense_stage(features, W1, b1.reshape(1, 64),
                              W2, b2.reshape(1, 64), W3, b3.reshape(1, 64),
                              W_fc)
    pa = _spmm_a_stage(adj3, vals3, p3, zero)
    pb, _ = _spmm_b_stage(adj3, vals3, p2, pa, zero)
    return _final_stage(p1, pb, bfc)


# speedup vs baseline: 17.9136x; 1.0489x over previous
import jax
import jax.numpy as jnp
from jax import lax
from jax.experimental import pallas as pl
from jax.experimental.pallas import tpu as pltpu
from jax.experimental.pallas import tpu_sc as plsc

N = 10000
E = 320000
D = 128
NC = 2
NS = 16
NW = NC * NS
EPW = E // NW
CH = 80
NCH = EPW // CH
RPS = N // NS

_f32 = jnp.float32
_i32 = jnp.int32

_SC_PARAMS = pltpu.CompilerParams(
    use_tc_tiling_on_sc=False, needs_layout_passes=False)


def _dense_body(x_ref, w1_ref, b1_ref, w2_ref, b2_ref, w3_ref, b3_ref,
                wfc_ref, p1_ref, p2_ref, p3_ref):
    x = x_ref[...]
    for k, (w_ref, b_ref, p_ref) in enumerate(
            [(w1_ref, b1_ref, p1_ref), (w2_ref, b2_ref, p2_ref),
             (w3_ref, b3_ref, p3_ref)]):
        h = jnp.dot(x, w_ref[...], preferred_element_type=_f32)
        h = jnp.maximum(h + b_ref[...], 0.0)
        f = wfc_ref[pl.ds(64 * k, 64), :]
        p_ref[...] = jnp.dot(h, f, preferred_element_type=_f32)


def _dense_stage(x, w1, b1, w2, b2, w3, b3, wfc):
    blk = 2000
    wspec = pl.BlockSpec((D, 64), lambda i: (0, 0))
    bspec = pl.BlockSpec((1, 64), lambda i: (0, 0))
    ospec = pl.BlockSpec((blk, 16), lambda i: (i, 0))
    oshape = jax.ShapeDtypeStruct((N, 16), _f32)
    return pl.pallas_call(
        _dense_body,
        grid=(N // blk,),
        in_specs=[
            pl.BlockSpec((blk, D), lambda i: (i, 0)),
            wspec, bspec, wspec, bspec, wspec, bspec,
            pl.BlockSpec((192, 16), lambda i: (0, 0)),
        ],
        out_specs=[ospec, ospec, ospec],
        out_shape=[oshape, oshape, oshape],
    )(x, w1, b1, w2, b2, w3, b3, wfc)


def _stage_edges(adj_hbm, vals_hbm, wid, rowv, colv, valv):
    pltpu.sync_copy(adj_hbm.at[0, wid], rowv)
    pltpu.sync_copy(adj_hbm.at[1, wid], colv)
    pltpu.sync_copy(vals_hbm.at[wid], valv)


def _edge_loop(tab_hbm, acc, rowv, colv, valv, g0, g1, sq0, sq1,
               gsem0, gsem1, ssem0, ssem1):
    def scale(j, g, sq):
        for i in range(CH):
            vi = plsc.load_gather(
                valv,
                [jnp.full((16,), j, _i32), jnp.full((16,), i, _i32)],
            )
            sq[i, :] = g[i, :] * vi

    def wait_gather(j, g, gsem):
        pltpu.make_async_copy(tab_hbm.at[colv.at[j]], g, gsem).wait()

    def start_gather(j, g, gsem):
        pltpu.async_copy(tab_hbm.at[colv.at[j]], g, gsem)

    def start_scatter(j, sq, ssem):
        pltpu.async_copy(sq, acc.at[rowv.at[j]], ssem, add=True)

    def wait_scatter(sq, ssem):
        pltpu.make_async_copy(sq, acc.at[rowv.at[0]], ssem).wait()

    def process(j, g, sq, gsem, ssem, first):
        wait_gather(j, g, gsem)
        if not first:
            wait_scatter(sq, ssem)
        scale(j, g, sq)
        start_scatter(j, sq, ssem)

    start_gather(0, g0, gsem0)
    start_gather(1, g1, gsem1)
    process(0, g0, sq0, gsem0, ssem0, True)
    start_gather(2, g0, gsem0)
    process(1, g1, sq1, gsem1, ssem1, True)
    start_gather(3, g1, gsem1)

    def pair(k, carry):
        j0 = 2 * k
        process(j0, g0, sq0, gsem0, ssem0, False)
        start_gather(j0 + 2, g0, gsem0)
        process(j0 + 1, g1, sq1, gsem1, ssem1, False)
        start_gather(j0 + 3, g1, gsem1)
        return carry

    lax.fori_loop(1, (NCH - 3) // 2, pair, 0)
    process(NCH - 3, g0, sq0, gsem0, ssem0, False)
    start_gather(NCH - 1, g0, gsem0)
    process(NCH - 2, g1, sq1, gsem1, ssem1, False)
    process(NCH - 1, g0, sq0, gsem0, ssem0, False)
    wait_scatter(sq0, ssem0)
    wait_scatter(sq1, ssem1)


def _spmm_a_body(adj_hbm, vals_hbm, tab_hbm, zero_hbm, out_hbm,
                 acc, rowv, colv, valv, g0, g1, sq0, sq1,
                 gsem0, gsem1, ssem0, ssem1):
    c = lax.axis_index("c")
    s = lax.axis_index("s")
    pltpu.sync_copy(zero_hbm.at[pl.ds(s * RPS, RPS)],
                    acc.at[pl.ds(s * RPS, RPS)])
    _stage_edges(adj_hbm, vals_hbm, c * NS + s, rowv, colv, valv)
    plsc.subcore_barrier()
    _edge_loop(tab_hbm, acc, rowv, colv, valv, g0, g1, sq0, sq1,
               gsem0, gsem1, ssem0, ssem1)
    plsc.subcore_barrier()
    pltpu.sync_copy(acc.at[pl.ds(s * RPS, RPS)],
                    out_hbm.at[c, pl.ds(s * RPS, RPS)])


def _spmm_b_body(adj_hbm, vals_hbm, p2_hbm, pa_hbm, zero_hbm,
                 out_hbm, u_hbm,
                 acc, rowv, colv, valv, g0, g1, sq0, sq1, ub, t0b, t1b,
                 gsem0, gsem1, ssem0, ssem1):
    c = lax.axis_index("c")
    s = lax.axis_index("s")
    sl = pl.ds(s * RPS, RPS)
    pltpu.sync_copy(p2_hbm.at[sl], ub)
    pltpu.sync_copy(pa_hbm.at[0, sl], t0b)
    pltpu.sync_copy(pa_hbm.at[1, sl], t1b)

    def add_row(r, carry):
        ub[r, :] = ub[r, :] + t0b[r, :] + t1b[r, :]
        return carry

    lax.fori_loop(0, RPS, add_row, 0)
    pltpu.sync_copy(ub, u_hbm.at[sl])
    pltpu.sync_copy(zero_hbm.at[sl], acc.at[sl])
    _stage_edges(adj_hbm, vals_hbm, c * NS + s, rowv, colv, valv)
    plsc.subcore_barrier()
    _edge_loop(u_hbm, acc, rowv, colv, valv, g0, g1, sq0, sq1,
               gsem0, gsem1, ssem0, ssem1)
    plsc.subcore_barrier()
    pltpu.sync_copy(acc.at[sl], out_hbm.at[c, sl])


_EDGE_SCRATCH = [
    pltpu.VMEM_SHARED((N, 16), _f32),
    pltpu.VMEM((NCH, CH), _i32),
    pltpu.VMEM((NCH, CH), _i32),
    pltpu.VMEM((NCH, CH), _f32),
    pltpu.VMEM((CH, 16), _f32),
    pltpu.VMEM((CH, 16), _f32),
    pltpu.VMEM((CH, 16), _f32),
    pltpu.VMEM((CH, 16), _f32),
]
_SEMS = [pltpu.SemaphoreType.DMA] * 4


def _spmm_a_stage(adj3, vals3, tab, zero):
    mesh = plsc.VectorSubcoreMesh(core_axis_name="c", subcore_axis_name="s")
    f = pl.kernel(
        _spmm_a_body,
        out_type=jax.ShapeDtypeStruct((NC, N, 16), _f32),
        mesh=mesh,
        compiler_params=_SC_PARAMS,
        scratch_types=_EDGE_SCRATCH + _SEMS,
    )
    return f(adj3, vals3, tab, zero)


def _spmm_b_stage(adj3, vals3, p2, pa, zero):
    mesh = plsc.VectorSubcoreMesh(core_axis_name="c", subcore_axis_name="s")
    f = pl.kernel(
        _spmm_b_body,
        out_type=[
            jax.ShapeDtypeStruct((NC, N, 16), _f32),
            jax.ShapeDtypeStruct((N, 16), _f32),
        ],
        mesh=mesh,
        compiler_params=_SC_PARAMS,
        scratch_types=_EDGE_SCRATCH + [
            pltpu.VMEM((RPS, 16), _f32),
            pltpu.VMEM((RPS, 16), _f32),
            pltpu.VMEM((RPS, 16), _f32),
        ] + _SEMS,
    )
    return f(adj3, vals3, p2, pa, zero)


def _final_body(p1_ref, qb_ref, bfc_ref, out_ref):
    logits = p1_ref[...] + qb_ref[0] + qb_ref[1] + bfc_ref[...]
    m = jnp.max(logits, axis=1, keepdims=True)
    sh = logits - m
    lse = jnp.log(jnp.sum(jnp.exp(sh), axis=1, keepdims=True))
    out_ref[...] = sh - lse


def _final_stage(p1, outb, bfc):
    blk = 2000
    return pl.pallas_call(
        _final_body,
        grid=(N // blk,),
        in_specs=[
            pl.BlockSpec((blk, 16), lambda i: (i, 0)),
            pl.BlockSpec((NC, blk, 16), lambda i: (0, i, 0)),
            pl.BlockSpec((1, 16), lambda i: (0, 0)),
        ],
        out_specs=pl.BlockSpec((blk, 16), lambda i: (i, 0)),
        out_shape=jax.ShapeDtypeStruct((N, 16), _f32),
    )(p1, outb, bfc)


def kernel(adj_indices, adj_values, features, W1, b1, W2, b2, W3, b3,
           W_fc, b_fc):
    adj3 = adj_indices.reshape(2, NW, NCH, CH)
    vals3 = adj_values.reshape(NW, NCH, CH)
    zero = jnp.zeros((N, 16), _f32)
    bfc = b_fc.reshape(1, 16)

    p1, p2, p3 = _dense_stage(features, W1, b1.reshape(1, 64),
                              W2, b2.reshape(1, 64), W3, b3.reshape(1, 64),
                              W_fc)
    pa = _spmm_a_stage(adj3, vals3, p3, zero)
    pb, _ = _spmm_b_stage(adj3, vals3, p2, pa, zero)
    return _final_stage(p1, pb, bfc)


# speedup vs baseline: 18.0044x; 1.0051x over previous
import jax
import jax.numpy as jnp
from jax import lax
from jax.experimental import pallas as pl
from jax.experimental.pallas import tpu as pltpu
from jax.experimental.pallas import tpu_sc as plsc

N = 10000
E = 320000
D = 128
NC = 2
NS = 16
NW = NC * NS
EPW = E // NW
CH = 80
NCH = EPW // CH
RPS = N // NS

_f32 = jnp.float32
_i32 = jnp.int32

_SC_PARAMS = pltpu.CompilerParams(
    use_tc_tiling_on_sc=False, needs_layout_passes=False)


def _branch(x, w_ref, b_ref, wfc_ref, k):
    h = jnp.dot(x, w_ref[...], preferred_element_type=_f32)
    h = jnp.maximum(h + b_ref[...], 0.0)
    f = wfc_ref[pl.ds(64 * k, 64), :]
    return jnp.dot(h, f, preferred_element_type=_f32)


def _dense3_body(x_ref, w3_ref, b3_ref, wfc_ref, p3_ref):
    p3_ref[...] = _branch(x_ref[...], w3_ref, b3_ref, wfc_ref, 2)


def _dense12_body(x_ref, w1_ref, b1_ref, w2_ref, b2_ref, wfc_ref,
                  p1_ref, p2_ref):
    x = x_ref[...]
    p1_ref[...] = _branch(x, w1_ref, b1_ref, wfc_ref, 0)
    p2_ref[...] = _branch(x, w2_ref, b2_ref, wfc_ref, 1)


_BLK = 2000
_XSPEC = pl.BlockSpec((_BLK, D), lambda i: (i, 0))
_WSPEC = pl.BlockSpec((D, 64), lambda i: (0, 0))
_BSPEC = pl.BlockSpec((1, 64), lambda i: (0, 0))
_FSPEC = pl.BlockSpec((192, 16), lambda i: (0, 0))
_OSPEC = pl.BlockSpec((_BLK, 16), lambda i: (i, 0))
_OSHAPE = jax.ShapeDtypeStruct((N, 16), _f32)


def _dense3_stage(x, w3, b3, wfc):
    return pl.pallas_call(
        _dense3_body,
        grid=(N // _BLK,),
        in_specs=[_XSPEC, _WSPEC, _BSPEC, _FSPEC],
        out_specs=_OSPEC,
        out_shape=_OSHAPE,
    )(x, w3, b3, wfc)


def _dense12_stage(x, w1, b1, w2, b2, wfc):
    return pl.pallas_call(
        _dense12_body,
        grid=(N // _BLK,),
        in_specs=[_XSPEC, _WSPEC, _BSPEC, _WSPEC, _BSPEC, _FSPEC],
        out_specs=[_OSPEC, _OSPEC],
        out_shape=[_OSHAPE, _OSHAPE],
    )(x, w1, b1, w2, b2, wfc)


def _zero_rows(buf, nrows):
    def z(r, carry):
        buf[r, :] = jnp.zeros((16,), _f32)
        return carry

    lax.fori_loop(0, nrows, z, 0)


def _stage_edges(adj_hbm, vals_hbm, wid, rowv, colv, valv, s0, s1, s2):
    base = wid * EPW
    d0 = pltpu.make_async_copy(adj_hbm.at[0, pl.ds(base, EPW)], rowv, s0)
    d1 = pltpu.make_async_copy(adj_hbm.at[1, pl.ds(base, EPW)], colv, s1)
    d2 = pltpu.make_async_copy(vals_hbm.at[pl.ds(base, EPW)], valv, s2)
    d0.start()
    d1.start()
    d2.start()
    d0.wait()
    d1.wait()
    d2.wait()


def _edge_loop(tab_hbm, acc, rowv, colv, valv, g0, g1, sq0, sq1,
               gsem0, gsem1, ssem0, ssem1):
    def scale(j, g, sq):
        for i in range(CH):
            vi = plsc.load_gather(valv, [jnp.full((16,), j * CH + i, _i32)])
            sq[i, :] = g[i, :] * vi

    def wait_gather(j, g, gsem):
        pltpu.make_async_copy(
            tab_hbm.at[colv.at[pl.ds(j * CH, CH)]], g, gsem).wait()

    def start_gather(j, g, gsem):
        pltpu.async_copy(tab_hbm.at[colv.at[pl.ds(j * CH, CH)]], g, gsem)

    def start_scatter(j, sq, ssem):
        pltpu.async_copy(sq, acc.at[rowv.at[pl.ds(j * CH, CH)]], ssem,
                         add=True)

    def wait_scatter(sq, ssem):
        pltpu.make_async_copy(sq, acc.at[rowv.at[pl.ds(0, CH)]], ssem).wait()

    def process(j, g, sq, gsem, ssem, first):
        wait_gather(j, g, gsem)
        if not first:
            wait_scatter(sq, ssem)
        scale(j, g, sq)
        start_scatter(j, sq, ssem)

    start_gather(0, g0, gsem0)
    start_gather(1, g1, gsem1)
    process(0, g0, sq0, gsem0, ssem0, True)
    start_gather(2, g0, gsem0)
    process(1, g1, sq1, gsem1, ssem1, True)
    start_gather(3, g1, gsem1)

    def pair(k, carry):
        j0 = 2 * k
        process(j0, g0, sq0, gsem0, ssem0, False)
        start_gather(j0 + 2, g0, gsem0)
        process(j0 + 1, g1, sq1, gsem1, ssem1, False)
        start_gather(j0 + 3, g1, gsem1)
        return carry

    lax.fori_loop(1, (NCH - 3) // 2, pair, 0)
    process(NCH - 3, g0, sq0, gsem0, ssem0, False)
    start_gather(NCH - 1, g0, gsem0)
    process(NCH - 2, g1, sq1, gsem1, ssem1, False)
    process(NCH - 1, g0, sq0, gsem0, ssem0, False)
    wait_scatter(sq0, ssem0)
    wait_scatter(sq1, ssem1)


def _spmm_a_body(adj_hbm, vals_hbm, tab_hbm, out_hbm,
                 acc, rowv, colv, valv, g0, g1, sq0, sq1, zb,
                 gsem0, gsem1, ssem0, ssem1):
    c = lax.axis_index("c")
    s = lax.axis_index("s")
    sl = pl.ds(s * RPS, RPS)
    _zero_rows(zb, RPS)
    pltpu.sync_copy(zb, acc.at[sl])
    _stage_edges(adj_hbm, vals_hbm, c * NS + s, rowv, colv, valv,
                 gsem0, gsem1, ssem0)
    plsc.subcore_barrier()
    _edge_loop(tab_hbm, acc, rowv, colv, valv, g0, g1, sq0, sq1,
               gsem0, gsem1, ssem0, ssem1)
    plsc.subcore_barrier()
    pltpu.sync_copy(acc.at[sl], out_hbm.at[c, sl])


def _spmm_b_body(adj_hbm, vals_hbm, p2_hbm, pa_hbm,
                 out_hbm, u_hbm,
                 acc, rowv, colv, valv, g0, g1, sq0, sq1, ub, t0b, t1b,
                 gsem0, gsem1, ssem0, ssem1):
    c = lax.axis_index("c")
    s = lax.axis_index("s")
    sl = pl.ds(s * RPS, RPS)
    d0 = pltpu.make_async_copy(p2_hbm.at[sl], ub, gsem0)
    d1 = pltpu.make_async_copy(pa_hbm.at[0, sl], t0b, gsem1)
    d2 = pltpu.make_async_copy(pa_hbm.at[1, sl], t1b, ssem0)
    d0.start()
    d1.start()
    d2.start()
    d0.wait()
    d1.wait()
    d2.wait()

    def add_row(r, carry):
        ub[r, :] = ub[r, :] + t0b[r, :] + t1b[r, :]
        return carry

    lax.fori_loop(0, RPS, add_row, 0)
    pltpu.sync_copy(ub, u_hbm.at[sl])
    _zero_rows(t0b, RPS)
    pltpu.sync_copy(t0b, acc.at[sl])
    _stage_edges(adj_hbm, vals_hbm, c * NS + s, rowv, colv, valv,
                 gsem0, gsem1, ssem0)
    plsc.subcore_barrier()
    _edge_loop(u_hbm, acc, rowv, colv, valv, g0, g1, sq0, sq1,
               gsem0, gsem1, ssem0, ssem1)
    plsc.subcore_barrier()
    pltpu.sync_copy(acc.at[sl], out_hbm.at[c, sl])


_BASE_SCRATCH = [
    pltpu.VMEM_SHARED((N, 16), _f32),
    pltpu.VMEM((EPW,), _i32),
    pltpu.VMEM((EPW,), _i32),
    pltpu.VMEM((EPW,), _f32),
    pltpu.VMEM((CH, 16), _f32),
    pltpu.VMEM((CH, 16), _f32),
    pltpu.VMEM((CH, 16), _f32),
    pltpu.VMEM((CH, 16), _f32),
]
_SEMS = [pltpu.SemaphoreType.DMA] * 4


def _spmm_a_stage(adj, vals, tab):
    mesh = plsc.VectorSubcoreMesh(core_axis_name="c", subcore_axis_name="s")
    f = pl.kernel(
        _spmm_a_body,
        out_type=jax.ShapeDtypeStruct((NC, N, 16), _f32),
        mesh=mesh,
        compiler_params=_SC_PARAMS,
        scratch_types=_BASE_SCRATCH + [pltpu.VMEM((RPS, 16), _f32)] + _SEMS,
    )
    return f(adj, vals, tab)


def _spmm_b_stage(adj, vals, p2, pa):
    mesh = plsc.VectorSubcoreMesh(core_axis_name="c", subcore_axis_name="s")
    f = pl.kernel(
        _spmm_b_body,
        out_type=[
            jax.ShapeDtypeStruct((NC, N, 16), _f32),
            jax.ShapeDtypeStruct((N, 16), _f32),
        ],
        mesh=mesh,
        compiler_params=_SC_PARAMS,
        scratch_types=_BASE_SCRATCH + [
            pltpu.VMEM((RPS, 16), _f32),
            pltpu.VMEM((RPS, 16), _f32),
            pltpu.VMEM((RPS, 16), _f32),
        ] + _SEMS,
    )
    return f(adj, vals, p2, pa)


def _final_body(p1_ref, qb_ref, bfc_ref, out_ref):
    logits = p1_ref[...] + qb_ref[0] + qb_ref[1] + bfc_ref[...]
    m = jnp.max(logits, axis=1, keepdims=True)
    sh = logits - m
    lse = jnp.log(jnp.sum(jnp.exp(sh), axis=1, keepdims=True))
    out_ref[...] = sh - lse


def _final_stage(p1, outb, bfc):
    return pl.pallas_call(
        _final_body,
        grid=(N // _BLK,),
        in_specs=[
            _OSPEC,
            pl.BlockSpec((NC, _BLK, 16), lambda i: (0, i, 0)),
            pl.BlockSpec((1, 16), lambda i: (0, 0)),
        ],
        out_specs=_OSPEC,
        out_shape=_OSHAPE,
    )(p1, outb, bfc)


def kernel(adj_indices, adj_values, features, W1, b1, W2, b2, W3, b3,
           W_fc, b_fc):
    bfc = b_fc.reshape(1, 16)
    p3 = _dense3_stage(features, W3, b3.reshape(1, 64), W_fc)
    pa = _spmm_a_stage(adj_indices, adj_values, p3)
    p1, p2 = _dense12_stage(features, W1, b1.reshape(1, 64),
                            W2, b2.reshape(1, 64), W_fc)
    pb, _ = _spmm_b_stage(adj_indices, adj_values, p2, pa)
    return _final_stage(p1, pb, bfc)


# speedup vs baseline: 22.1155x; 1.2283x over previous
import jax
import jax.numpy as jnp
from jax import lax
from jax.experimental import pallas as pl
from jax.experimental.pallas import tpu as pltpu
from jax.experimental.pallas import tpu_sc as plsc

N = 10000
E = 320000
D = 128
NC = 2
NS = 16
NW = NC * NS
EPW = E // NW
CH = 80
NCH = EPW // CH
RPS = N // NS

_f32 = jnp.float32
_i32 = jnp.int32

_SC_PARAMS = pltpu.CompilerParams(
    use_tc_tiling_on_sc=False, needs_layout_passes=False)


def _branch(x, w_ref, b_ref, wfc_ref, k):
    h = jnp.dot(x, w_ref[...], preferred_element_type=_f32)
    h = jnp.maximum(h + b_ref[...], 0.0)
    f = wfc_ref[pl.ds(64 * k, 64), :]
    return jnp.dot(h, f, preferred_element_type=_f32)


def _dense3_body(x_ref, w3_ref, b3_ref, wfc_ref, p3_ref):
    p3_ref[...] = _branch(x_ref[...], w3_ref, b3_ref, wfc_ref, 2)


def _dense12_body(x_ref, w1_ref, b1_ref, w2_ref, b2_ref, wfc_ref,
                  p1_ref, p2_ref):
    x = x_ref[...]
    p1_ref[...] = _branch(x, w1_ref, b1_ref, wfc_ref, 0)
    p2_ref[...] = _branch(x, w2_ref, b2_ref, wfc_ref, 1)


_BLK = 2000
_XSPEC = pl.BlockSpec((_BLK, D), lambda i: (i, 0))
_WSPEC = pl.BlockSpec((D, 64), lambda i: (0, 0))
_BSPEC = pl.BlockSpec((1, 64), lambda i: (0, 0))
_FSPEC = pl.BlockSpec((192, 16), lambda i: (0, 0))
_OSPEC = pl.BlockSpec((_BLK, 16), lambda i: (i, 0))
_OSHAPE = jax.ShapeDtypeStruct((N, 16), _f32)


def _dense3_stage(x, w3, b3, wfc):
    return pl.pallas_call(
        _dense3_body,
        grid=(N // _BLK,),
        in_specs=[_XSPEC, _WSPEC, _BSPEC, _FSPEC],
        out_specs=_OSPEC,
        out_shape=_OSHAPE,
    )(x, w3, b3, wfc)


def _dense12_stage(x, w1, b1, w2, b2, wfc):
    return pl.pallas_call(
        _dense12_body,
        grid=(N // _BLK,),
        in_specs=[_XSPEC, _WSPEC, _BSPEC, _WSPEC, _BSPEC, _FSPEC],
        out_specs=[_OSPEC, _OSPEC],
        out_shape=[_OSHAPE, _OSHAPE],
    )(x, w1, b1, w2, b2, wfc)


def _zero_rows(buf, nrows):
    def z(r, carry):
        buf[r, :] = jnp.zeros((16,), _f32)
        return carry

    lax.fori_loop(0, nrows, z, 0)


def _stage_edges(adj_hbm, vals_hbm, wid, rowv, colv, valv, s0, s1, s2):
    base = wid * EPW
    d0 = pltpu.make_async_copy(adj_hbm.at[0, pl.ds(base, EPW)], rowv, s0)
    d1 = pltpu.make_async_copy(adj_hbm.at[1, pl.ds(base, EPW)], colv, s1)
    d2 = pltpu.make_async_copy(vals_hbm.at[pl.ds(base, EPW)], valv, s2)
    d0.start()
    d1.start()
    d2.start()
    d0.wait()
    d1.wait()
    d2.wait()


def _edge_loop(tab_hbm, acc, rowv, colv, valv, g0, g1, sq0, sq1,
               gsem0, gsem1, ssem0, ssem1):
    def scale(j, g, sq):
        for i2 in range(CH // 16):
            vv = valv[pl.ds(j * CH + i2 * 16, 16)]
            for e in range(16):
                i = i2 * 16 + e
                vi = vv.at[jnp.full((16,), e, _i32)].get(
                    mode="promise_in_bounds")
                sq[i, :] = g[i, :] * vi

    def wait_gather(j, g, gsem):
        pltpu.make_async_copy(
            tab_hbm.at[colv.at[pl.ds(j * CH, CH)]], g, gsem).wait()

    def start_gather(j, g, gsem):
        pltpu.async_copy(tab_hbm.at[colv.at[pl.ds(j * CH, CH)]], g, gsem)

    def start_scatter(j, sq, ssem):
        pltpu.async_copy(sq, acc.at[rowv.at[pl.ds(j * CH, CH)]], ssem,
                         add=True)

    def wait_scatter(sq, ssem):
        pltpu.make_async_copy(sq, acc.at[rowv.at[pl.ds(0, CH)]], ssem).wait()

    def process(j, g, sq, gsem, ssem, first):
        wait_gather(j, g, gsem)
        if not first:
            wait_scatter(sq, ssem)
        scale(j, g, sq)
        start_scatter(j, sq, ssem)

    start_gather(0, g0, gsem0)
    start_gather(1, g1, gsem1)
    process(0, g0, sq0, gsem0, ssem0, True)
    start_gather(2, g0, gsem0)
    process(1, g1, sq1, gsem1, ssem1, True)
    start_gather(3, g1, gsem1)

    def pair(k, carry):
        j0 = 2 * k
        process(j0, g0, sq0, gsem0, ssem0, False)
        start_gather(j0 + 2, g0, gsem0)
        process(j0 + 1, g1, sq1, gsem1, ssem1, False)
        start_gather(j0 + 3, g1, gsem1)
        return carry

    lax.fori_loop(1, (NCH - 3) // 2, pair, 0)
    process(NCH - 3, g0, sq0, gsem0, ssem0, False)
    start_gather(NCH - 1, g0, gsem0)
    process(NCH - 2, g1, sq1, gsem1, ssem1, False)
    process(NCH - 1, g0, sq0, gsem0, ssem0, False)
    wait_scatter(sq0, ssem0)
    wait_scatter(sq1, ssem1)


def _spmm_a_body(adj_hbm, vals_hbm, tab_hbm, out_hbm,
                 acc, rowv, colv, valv, g0, g1, sq0, sq1, zb,
                 gsem0, gsem1, ssem0, ssem1):
    c = lax.axis_index("c")
    s = lax.axis_index("s")
    sl = pl.ds(s * RPS, RPS)
    _zero_rows(zb, RPS)
    pltpu.sync_copy(zb, acc.at[sl])
    _stage_edges(adj_hbm, vals_hbm, c * NS + s, rowv, colv, valv,
                 gsem0, gsem1, ssem0)
    plsc.subcore_barrier()
    _edge_loop(tab_hbm, acc, rowv, colv, valv, g0, g1, sq0, sq1,
               gsem0, gsem1, ssem0, ssem1)
    plsc.subcore_barrier()
    pltpu.sync_copy(acc.at[sl], out_hbm.at[c, sl])


def _spmm_b_body(adj_hbm, vals_hbm, p2_hbm, pa_hbm,
                 out_hbm, u_hbm,
                 acc, rowv, colv, valv, g0, g1, sq0, sq1, ub, t0b, t1b,
                 gsem0, gsem1, ssem0, ssem1):
    c = lax.axis_index("c")
    s = lax.axis_index("s")
    sl = pl.ds(s * RPS, RPS)
    base = (c * NS + s) * EPW
    e0 = pltpu.make_async_copy(adj_hbm.at[0, pl.ds(base, EPW)], rowv, ssem1)
    e1 = pltpu.make_async_copy(adj_hbm.at[1, pl.ds(base, EPW)], colv, gsem1)
    e2 = pltpu.make_async_copy(vals_hbm.at[pl.ds(base, EPW)], valv, ssem0)
    e0.start()
    e1.start()
    e2.start()
    pltpu.sync_copy(p2_hbm.at[sl], ub)
    pltpu.sync_copy(pa_hbm.at[0, sl], t0b)
    pltpu.sync_copy(pa_hbm.at[1, sl], t1b)

    def add_row(r, carry):
        ub[r, :] = ub[r, :] + t0b[r, :] + t1b[r, :]
        return carry

    lax.fori_loop(0, RPS, add_row, 0)
    pltpu.sync_copy(ub, u_hbm.at[sl])
    _zero_rows(t0b, RPS)
    pltpu.sync_copy(t0b, acc.at[sl])
    e0.wait()
    e1.wait()
    e2.wait()
    plsc.subcore_barrier()
    _edge_loop(u_hbm, acc, rowv, colv, valv, g0, g1, sq0, sq1,
               gsem0, gsem1, ssem0, ssem1)
    plsc.subcore_barrier()
    pltpu.sync_copy(acc.at[sl], out_hbm.at[c, sl])


_BASE_SCRATCH = [
    pltpu.VMEM_SHARED((N, 16), _f32),
    pltpu.VMEM((EPW,), _i32),
    pltpu.VMEM((EPW,), _i32),
    pltpu.VMEM((EPW,), _f32),
    pltpu.VMEM((CH, 16), _f32),
    pltpu.VMEM((CH, 16), _f32),
    pltpu.VMEM((CH, 16), _f32),
    pltpu.VMEM((CH, 16), _f32),
]
_SEMS = [pltpu.SemaphoreType.DMA] * 4


def _spmm_a_stage(adj, vals, tab):
    mesh = plsc.VectorSubcoreMesh(core_axis_name="c", subcore_axis_name="s")
    f = pl.kernel(
        _spmm_a_body,
        out_type=jax.ShapeDtypeStruct((NC, N, 16), _f32),
        mesh=mesh,
        compiler_params=_SC_PARAMS,
        scratch_types=_BASE_SCRATCH + [pltpu.VMEM((RPS, 16), _f32)] + _SEMS,
    )
    return f(adj, vals, tab)


def _spmm_b_stage(adj, vals, p2, pa):
    mesh = plsc.VectorSubcoreMesh(core_axis_name="c", subcore_axis_name="s")
    f = pl.kernel(
        _spmm_b_body,
        out_type=[
            jax.ShapeDtypeStruct((NC, N, 16), _f32),
            jax.ShapeDtypeStruct((N, 16), _f32),
        ],
        mesh=mesh,
        compiler_params=_SC_PARAMS,
        scratch_types=_BASE_SCRATCH + [
            pltpu.VMEM((RPS, 16), _f32),
            pltpu.VMEM((RPS, 16), _f32),
            pltpu.VMEM((RPS, 16), _f32),
        ] + _SEMS,
    )
    return f(adj, vals, p2, pa)


def _final_body(p1_ref, qb_ref, bfc_ref, out_ref):
    logits = p1_ref[...] + qb_ref[0] + qb_ref[1] + bfc_ref[...]
    m = jnp.max(logits, axis=1, keepdims=True)
    sh = logits - m
    lse = jnp.log(jnp.sum(jnp.exp(sh), axis=1, keepdims=True))
    out_ref[...] = sh - lse


def _final_stage(p1, outb, bfc):
    return pl.pallas_call(
        _final_body,
        grid=(N // _BLK,),
        in_specs=[
            _OSPEC,
            pl.BlockSpec((NC, _BLK, 16), lambda i: (0, i, 0)),
            pl.BlockSpec((1, 16), lambda i: (0, 0)),
        ],
        out_specs=_OSPEC,
        out_shape=_OSHAPE,
    )(p1, outb, bfc)


def kernel(adj_indices, adj_values, features, W1, b1, W2, b2, W3, b3,
           W_fc, b_fc):
    bfc = b_fc.reshape(1, 16)
    p3 = _dense3_stage(features, W3, b3.reshape(1, 64), W_fc)
    pa = _spmm_a_stage(adj_indices, adj_values, p3)
    p1, p2 = _dense12_stage(features, W1, b1.reshape(1, 64),
                            W2, b2.reshape(1, 64), W_fc)
    pb, _ = _spmm_b_stage(adj_indices, adj_values, p2, pa)
    return _final_stage(p1, pb, bfc)
